# Initial kernel scaffold; baseline (speedup 1.0000x reference)
import functools
import math

import jax
import jax.numpy as jnp
from jax import lax
from jax.experimental import pallas as pl
from jax.experimental.pallas import tpu as pltpu
from jax.experimental.pallas import tpu_sc as plsc

_N = 50000
_E = 800000
_D = 64
_ED = 16
_G = 512

_EP = 802816
_ECH = _EP // 128
_BE = 2048
_BN = 2000
_ACC_R = 51200
_NP = 53248
_NCH = _NP // 128

_mesh = plsc.VectorSubcoreMesh(core_axis_name="c", subcore_axis_name="s")


def _leaky(x):
    return jnp.where(x >= 0, x, 0.15 * x)


def _sigmoid(x):
    return 1.0 / (1.0 + jnp.exp(-x))


def _ln(x, g, b):
    m = jnp.mean(x, axis=-1, keepdims=True)
    v = jnp.mean((x - m) * (x - m), axis=-1, keepdims=True)
    return (x - m) / jnp.sqrt(v + 1e-5) * g + b


def _make_sc_gather(w, kb):
    n_ch_tile = _ECH // 32
    n_sup = n_ch_tile // kb

    @functools.partial(
        pl.kernel,
        mesh=_mesh,
        out_type=jax.ShapeDtypeStruct((_EP, w), jnp.float32),
        scratch_types=[
            pltpu.VMEM((kb, 128), jnp.int32),
            pltpu.VMEM((kb * 128, w), jnp.float32),
            pltpu.SemaphoreType.DMA,
        ],
    )
    def k(table_hbm, idx_hbm, out_hbm, idx_v, rows_v, sem):
        c = lax.axis_index("c")
        s = lax.axis_index("s")
        ch_base = (c * 16 + s) * n_ch_tile

        def body(sup, carry):
            ch0 = ch_base + sup * kb
            pltpu.sync_copy(idx_hbm.at[pl.ds(ch0, kb)], idx_v)
            handles = [
                pltpu.async_copy(
                    table_hbm.at[idx_v.at[j]],
                    rows_v.at[pl.ds(j * 128, 128)],
                    sem,
                )
                for j in range(kb)
            ]
            for h in handles:
                h.wait()
            pltpu.sync_copy(rows_v, out_hbm.at[pl.ds(ch0 * 128, kb * 128)])
            return carry

        lax.fori_loop(0, n_sup, body, 0)

    return k


def _make_sc_scatter64():
    kb = 8
    n_ch_tile = _ECH // 16
    n_sup = n_ch_tile // kb
    zrows = _ACC_R // 16
    orows = _N // 16

    @functools.partial(
        pl.kernel,
        mesh=_mesh,
        out_type=jax.ShapeDtypeStruct((_N, 64), jnp.float32),
        scratch_types=[
            pltpu.VMEM((kb, 128), jnp.int32),
            pltpu.VMEM((kb * 128, 32), jnp.float32),
            pltpu.VMEM_SHARED((_ACC_R, 32), jnp.float32),
            pltpu.SemaphoreType.DMA,
        ],
    )
    def k(vals_hbm, idx_hbm, zeros_hbm, out_hbm, idx_v, vals_v, acc, sem):
        c = lax.axis_index("c")
        s = lax.axis_index("s")
        pltpu.sync_copy(
            zeros_hbm.at[pl.ds(0, zrows), pl.ds(0, 32)],
            acc.at[pl.ds(s * zrows, zrows)],
        )
        plsc.subcore_barrier()

        def body(sup, carry):
            ch0 = s * n_ch_tile + sup * kb
            pltpu.sync_copy(idx_hbm.at[pl.ds(ch0, kb)], idx_v)
            pltpu.sync_copy(
                vals_hbm.at[pl.ds(ch0 * 128, kb * 128), pl.ds(c * 32, 32)],
                vals_v,
            )
            handles = [
                pltpu.async_copy(
                    vals_v.at[pl.ds(j * 128, 128)],
                    acc.at[idx_v.at[j]],
                    sem,
                    add=True,
                )
                for j in range(kb)
            ]
            for h in handles:
                h.wait()
            return carry

        lax.fori_loop(0, n_sup, body, 0)
        plsc.subcore_barrier()
        pltpu.sync_copy(
            acc.at[pl.ds(s * orows, orows)],
            out_hbm.at[pl.ds(s * orows, orows), pl.ds(c * 32, 32)],
        )

    return k


def _make_sc_scatter16():
    kb = 7
    n_ch_tile = _ECH // 32
    n_sup = n_ch_tile // kb
    zrows = _ACC_R // 16
    orows = _N // 16

    @functools.partial(
        pl.kernel,
        mesh=_mesh,
        out_type=jax.ShapeDtypeStruct((2 * _N, 16), jnp.float32),
        scratch_types=[
            pltpu.VMEM((kb, 128), jnp.int32),
            pltpu.VMEM((kb * 128, 16), jnp.float32),
            pltpu.VMEM_SHARED((_ACC_R, 16), jnp.float32),
            pltpu.SemaphoreType.DMA,
        ],
    )
    def k(vals_hbm, idx_hbm, zeros_hbm, out_hbm, idx_v, vals_v, acc, sem):
        c = lax.axis_index("c")
        s = lax.axis_index("s")
        pltpu.sync_copy(
            zeros_hbm.at[pl.ds(0, zrows), pl.ds(0, 16)],
            acc.at[pl.ds(s * zrows, zrows)],
        )
        plsc.subcore_barrier()

        def body(sup, carry):
            ch0 = (c * 16 + s) * n_ch_tile + sup * kb
            pltpu.sync_copy(idx_hbm.at[pl.ds(ch0, kb)], idx_v)
            pltpu.sync_copy(vals_hbm.at[pl.ds(ch0 * 128, kb * 128)], vals_v)
            handles = [
                pltpu.async_copy(
                    vals_v.at[pl.ds(j * 128, 128)],
                    acc.at[idx_v.at[j]],
                    sem,
                    add=True,
                )
                for j in range(kb)
            ]
            for h in handles:
                h.wait()
            return carry

        lax.fori_loop(0, n_sup, body, 0)
        plsc.subcore_barrier()
        pltpu.sync_copy(
            acc.at[pl.ds(s * orows, orows)],
            out_hbm.at[pl.ds(c * _N + s * orows, orows)],
        )

    return k


def _make_sc_pool():
    kb = 13
    n_ch_tile = _NCH // 32
    gacc = _G + 16
    zrows = gacc // 16
    orows = _G // 16

    @functools.partial(
        pl.kernel,
        mesh=_mesh,
        out_type=[
            jax.ShapeDtypeStruct((2 * _G, 64), jnp.float32),
            jax.ShapeDtypeStruct((2 * _G, 16), jnp.float32),
        ],
        scratch_types=[
            pltpu.VMEM((kb, 128), jnp.int32),
            pltpu.VMEM((kb * 128, 64), jnp.float32),
            pltpu.VMEM((128, 16), jnp.float32),
            pltpu.VMEM_SHARED((gacc, 64), jnp.float32),
            pltpu.VMEM_SHARED((gacc, 16), jnp.float32),
            pltpu.SemaphoreType.DMA,
        ],
    )
    def k(h_hbm, idx_hbm, zeros_hbm, ones_hbm, outh_hbm, outc_hbm,
          idx_v, vals_v, ones_v, acc_h, acc_c, sem):
        c = lax.axis_index("c")
        s = lax.axis_index("s")
        pltpu.sync_copy(ones_hbm, ones_v)
        pltpu.sync_copy(
            zeros_hbm.at[pl.ds(0, zrows), pl.ds(0, 64)],
            acc_h.at[pl.ds(s * zrows, zrows)],
        )
        pltpu.sync_copy(
            zeros_hbm.at[pl.ds(0, zrows), pl.ds(0, 16)],
            acc_c.at[pl.ds(s * zrows, zrows)],
        )
        plsc.subcore_barrier()

        ch0 = (c * 16 + s) * n_ch_tile
        pltpu.sync_copy(idx_hbm.at[pl.ds(ch0, kb)], idx_v)
        pltpu.sync_copy(h_hbm.at[pl.ds(ch0 * 128, kb * 128)], vals_v)
        handles = []
        for j in range(kb):
            handles.append(pltpu.async_copy(
                vals_v.at[pl.ds(j * 128, 128)],
                acc_h.at[idx_v.at[j]],
                sem,
                add=True,
            ))
            handles.append(pltpu.async_copy(
                ones_v,
                acc_c.at[idx_v.at[j]],
                sem,
                add=True,
            ))
        for h in handles:
            h.wait()
        plsc.subcore_barrier()
        pltpu.sync_copy(
            acc_h.at[pl.ds(s * orows, orows)],
            outh_hbm.at[pl.ds(c * _G + s * orows, orows)],
        )
        pltpu.sync_copy(
            acc_c.at[pl.ds(s * orows, orows)],
            outc_hbm.at[pl.ds(c * _G + s * orows, orows)],
        )

    return k


_sc_gather64 = _make_sc_gather(64, 7)
_sc_gather128 = _make_sc_gather(128, 4)
_sc_scatter64 = _make_sc_scatter64()
_sc_scatter16 = _make_sc_scatter16()
_sc_pool = _make_sc_pool()


def _full_spec(shape):
    nd = len(shape)
    return pl.BlockSpec(shape, lambda i=0, _n=nd: (0,) * _n)


def _tc_in(x, w, b):
    def body(x_ref, w_ref, b_ref, o_ref):
        o_ref[...] = x_ref[...] * w_ref[...] + b_ref[...]

    return pl.pallas_call(
        body,
        grid=(_N // _BN,),
        in_specs=[
            pl.BlockSpec((_BN, 1), lambda i: (i, 0)),
            _full_spec((1, _D)),
            _full_spec((1, _D)),
        ],
        out_specs=pl.BlockSpec((_BN, _D), lambda i: (i, 0)),
        out_shape=jax.ShapeDtypeStruct((_N, _D), jnp.float32),
    )(x, w, b)


def _tc_gin_edge(ea, xg, p):
    def body(ea_ref, xg_ref, w1, b1, lg, lb, w2, b2, w3, b3, o_ref):
        e = jnp.dot(ea_ref[...], w1[...], preferred_element_type=jnp.float32)
        e = _ln(e + b1[...], lg[...], lb[...])
        e = _leaky(e)
        e = _leaky(jnp.dot(e, w2[...], preferred_element_type=jnp.float32) + b2[...])
        e = jnp.dot(e, w3[...], preferred_element_type=jnp.float32) + b3[...]
        gate = _sigmoid(e)
        o_ref[...] = gate * xg_ref[...] + (1.0 - gate) * e

    enc = p['enc']
    return pl.pallas_call(
        body,
        grid=(_EP // _BE,),
        in_specs=[
            pl.BlockSpec((_BE, _ED), lambda i: (i, 0)),
            pl.BlockSpec((_BE, _D), lambda i: (i, 0)),
            _full_spec((_ED, _D)), _full_spec((1, _D)),
            _full_spec((1, _D)), _full_spec((1, _D)),
            _full_spec((_D, _D)), _full_spec((1, _D)),
            _full_spec((_D, _D)), _full_spec((1, _D)),
        ],
        out_specs=pl.BlockSpec((_BE, _D), lambda i: (i, 0)),
        out_shape=jax.ShapeDtypeStruct((_EP, _D), jnp.float32),
    )(ea, xg,
      enc['w1'], enc['b1'].reshape(1, -1),
      enc['ln_g'].reshape(1, -1), enc['ln_b'].reshape(1, -1),
      enc['w2'], enc['b2'].reshape(1, -1),
      enc['w3'], enc['b3'].reshape(1, -1))


def _tc_gin_node(h, aggr, p):
    def body(h_ref, a_ref, eps, w1, b1, l1g, l1b, w2, b2, l2g, l2b, o_ref):
        x = h_ref[...]
        t = (1.0 + eps[0, 0]) * x + a_ref[...]
        t = jnp.dot(t, w1[...], preferred_element_type=jnp.float32) + b1[...]
        t = _leaky(_ln(t, l1g[...], l1b[...]))
        t = jnp.dot(t, w2[...], preferred_element_type=jnp.float32) + b2[...]
        t = _ln(t, l2g[...], l2b[...])
        o_ref[...] = x + t

    m = p['mlp']
    return pl.pallas_call(
        body,
        grid=(_N // _BN,),
        in_specs=[
            pl.BlockSpec((_BN, _D), lambda i: (i, 0)),
            pl.BlockSpec((_BN, _D), lambda i: (i, 0)),
            _full_spec((1, 1)),
            _full_spec((_D, 2 * _D)), _full_spec((1, 2 * _D)),
            _full_spec((1, 2 * _D)), _full_spec((1, 2 * _D)),
            _full_spec((2 * _D, _D)), _full_spec((1, _D)),
            _full_spec((1, _D)), _full_spec((1, _D)),
        ],
        out_specs=pl.BlockSpec((_BN, _D), lambda i: (i, 0)),
        out_shape=jax.ShapeDtypeStruct((_N, _D), jnp.float32),
    )(h, aggr, p['eps'].reshape(1, 1),
      m['w1'], m['b1'].reshape(1, -1), m['ln1_g'].reshape(1, -1),
      m['ln1_b'].reshape(1, -1), m['w2'], m['b2'].reshape(1, -1),
      m['ln2_g'].reshape(1, -1), m['ln2_b'].reshape(1, -1))


def _tc_bn_leaky(h, g, b):
    def body(h_ref, g_ref, b_ref, o_ref):
        x = h_ref[...]
        m = jnp.sum(x, axis=0, keepdims=True) * (1.0 / _N)
        d = x - m
        v = jnp.sum(d * d, axis=0, keepdims=True) * (1.0 / _N)
        o_ref[...] = _leaky(d / jnp.sqrt(v + 1e-5) * g_ref[...] + b_ref[...])

    return pl.pallas_call(
        body,
        in_specs=[_full_spec((_N, _D)), _full_spec((1, _D)), _full_spec((1, _D))],
        out_specs=_full_spec((_N, _D)),
        out_shape=jax.ShapeDtypeStruct((_N, _D), jnp.float32),
    )(h, g.reshape(1, -1), b.reshape(1, -1))


def _tc_bn_leaky_res(h, g, b, res):
    def body(h_ref, g_ref, b_ref, r_ref, o_ref):
        x = h_ref[...]
        m = jnp.sum(x, axis=0, keepdims=True) * (1.0 / _N)
        d = x - m
        v = jnp.sum(d * d, axis=0, keepdims=True) * (1.0 / _N)
        o_ref[...] = (_leaky(d / jnp.sqrt(v + 1e-5) * g_ref[...] + b_ref[...])
                      + r_ref[...])

    return pl.pallas_call(
        body,
        in_specs=[_full_spec((_N, _D)), _full_spec((1, _D)), _full_spec((1, _D)),
                  _full_spec((_N, _D))],
        out_specs=_full_spec((_N, _D)),
        out_shape=jax.ShapeDtypeStruct((_N, _D), jnp.float32),
    )(h, g.reshape(1, -1), b.reshape(1, -1), res)


def _tc_qkv(h, p):
    def body(h_ref, wq, bq, wk, bk, wv, bv, q_ref, kv_ref):
        x = h_ref[...]
        q_ref[...] = jnp.dot(x, wq[...], preferred_element_type=jnp.float32) + bq[...]
        kv_ref[:, 0:_D] = jnp.dot(x, wk[...], preferred_element_type=jnp.float32) + bk[...]
        kv_ref[:, _D:2 * _D] = jnp.dot(x, wv[...], preferred_element_type=jnp.float32) + bv[...]

    return pl.pallas_call(
        body,
        grid=(_N // _BN,),
        in_specs=[
            pl.BlockSpec((_BN, _D), lambda i: (i, 0)),
            _full_spec((_D, _D)), _full_spec((1, _D)),
            _full_spec((_D, _D)), _full_spec((1, _D)),
            _full_spec((_D, _D)), _full_spec((1, _D)),
        ],
        out_specs=[
            pl.BlockSpec((_BN, _D), lambda i: (i, 0)),
            pl.BlockSpec((_BN, 2 * _D), lambda i: (i, 0)),
        ],
        out_shape=[
            jax.ShapeDtypeStruct((_N, _D), jnp.float32),
            jax.ShapeDtypeStruct((_N, 2 * _D), jnp.float32),
        ],
    )(h, p['wq'], p['bq'].reshape(1, -1), p['wk'], p['bk'].reshape(1, -1),
      p['wv'], p['bv'].reshape(1, -1))


def _tc_tr_edge(ea, kvg, qg, p):
    isq = 1.0 / math.sqrt(_D // 2)

    def body(ea_ref, kvg_ref, qg_ref, w1, b1, w2, b2, we, be, num_ref, ue_ref):
        e = _leaky(jnp.dot(ea_ref[...], w1[...], preferred_element_type=jnp.float32) + b1[...])
        e = jnp.dot(e, w2[...], preferred_element_type=jnp.float32) + b2[...]
        ek = jnp.dot(e, we[...], preferred_element_type=jnp.float32) + be[...]
        kj = kvg_ref[:, 0:_D] + ek
        qk = qg_ref[...] * kj
        a0 = jnp.sum(qk[:, 0:32], axis=1, keepdims=True) * isq
        a1 = jnp.sum(qk[:, 32:64], axis=1, keepdims=True) * isq
        ue0 = jnp.exp(a0)
        ue1 = jnp.exp(a1)
        col = lax.broadcasted_iota(jnp.int32, (_BE, _D), 1)
        ueb = jnp.where(col < 32, ue0, ue1)
        num_ref[...] = (kvg_ref[:, _D:2 * _D] + ek) * ueb
        col16 = lax.broadcasted_iota(jnp.int32, (_BE, 16), 1)
        ue_ref[...] = jnp.where(col16 < 8, ue0, ue1)

    return pl.pallas_call(
        body,
        grid=(_EP // _BE,),
        in_specs=[
            pl.BlockSpec((_BE, _ED), lambda i: (i, 0)),
            pl.BlockSpec((_BE, 2 * _D), lambda i: (i, 0)),
            pl.BlockSpec((_BE, _D), lambda i: (i, 0)),
            _full_spec((_ED, _D)), _full_spec((1, _D)),
            _full_spec((_D, _D)), _full_spec((1, _D)),
            _full_spec((_D, _D)), _full_spec((1, _D)),
        ],
        out_specs=[
            pl.BlockSpec((_BE, _D), lambda i: (i, 0)),
            pl.BlockSpec((_BE, 16), lambda i: (i, 0)),
        ],
        out_shape=[
            jax.ShapeDtypeStruct((_EP, _D), jnp.float32),
            jax.ShapeDtypeStruct((_EP, 16), jnp.float32),
        ],
    )(ea, kvg, qg,
      p['enc_w1'], p['enc_b1'].reshape(1, -1),
      p['enc_w2'], p['enc_b2'].reshape(1, -1),
      p['we'], p['be'].reshape(1, -1))


def _tc_tr_node(h, num, den_a, den_b, p):
    wb = p['wbeta']

    def body(h_ref, n_ref, da_ref, db_ref, wskip, bskip, wb0, wb1, wb2, o_ref):
        den = da_ref[...] + db_ref[...]
        d0 = den[:, 0:1]
        d1 = den[:, 8:9]
        num = n_ref[...]
        out0 = num[:, 0:32] / (d0 + 1e-16)
        out1 = num[:, 32:64] / (d1 + 1e-16)
        out = jnp.concatenate([out0, out1], axis=1)
        x_r = jnp.dot(h_ref[...], wskip[...], preferred_element_type=jnp.float32) + bskip[...]
        bl = (jnp.dot(out, wb0[...], preferred_element_type=jnp.float32)
              + jnp.dot(x_r, wb1[...], preferred_element_type=jnp.float32)
              + jnp.dot(out - x_r, wb2[...], preferred_element_type=jnp.float32))
        beta = _sigmoid(bl)
        o_ref[...] = beta * x_r + (1.0 - beta) * out

    return pl.pallas_call(
        body,
        grid=(_N // _BN,),
        in_specs=[
            pl.BlockSpec((_BN, _D), lambda i: (i, 0)),
            pl.BlockSpec((_BN, _D), lambda i: (i, 0)),
            pl.BlockSpec((_BN, 16), lambda i: (i, 0)),
            pl.BlockSpec((_BN, 16), lambda i: (i, 0)),
            _full_spec((_D, _D)), _full_spec((1, _D)),
            _full_spec((_D, 1)), _full_spec((_D, 1)), _full_spec((_D, 1)),
        ],
        out_specs=pl.BlockSpec((_BN, _D), lambda i: (i, 0)),
        out_shape=jax.ShapeDtypeStruct((_N, _D), jnp.float32),
    )(h, num, den_a, den_b, p['wskip'], p['bskip'].reshape(1, -1),
      wb[0:_D], wb[_D:2 * _D], wb[2 * _D:3 * _D])


def _tc_cls(sa, sb, ca, cb, w1, b1, w2, b2):
    def body(sa_ref, sb_ref, ca_ref, cb_ref, w1r, b1r, w2r, b2r, o_ref):
        sums = sa_ref[...] + sb_ref[...]
        cnt = ca_ref[:, 0:1] + cb_ref[:, 0:1]
        pooled = sums / jnp.maximum(cnt, 1.0)
        z = _leaky(jnp.dot(pooled, w1r[...], preferred_element_type=jnp.float32) + b1r[...])
        o_ref[...] = jnp.dot(z, w2r[...], preferred_element_type=jnp.float32) + b2r[...]

    return pl.pallas_call(
        body,
        in_specs=[
            _full_spec((_G, _D)), _full_spec((_G, _D)),
            _full_spec((_G, 16)), _full_spec((_G, 16)),
            _full_spec((_D, _D // 2)), _full_spec((1, _D // 2)),
            _full_spec((_D // 2, 10)), _full_spec((1, 10)),
        ],
        out_specs=_full_spec((_G, 10)),
        out_shape=jax.ShapeDtypeStruct((_G, 10), jnp.float32),
    )(sa, sb, ca, cb, w1, b1.reshape(1, -1), w2, b2.reshape(1, -1))


def _gin_layer(p, h, ea, src2, dst2, zeros):
    xg = _sc_gather64(h, src2)
    msg = _tc_gin_edge(ea, xg, p)
    aggr = _sc_scatter64(msg, dst2, zeros)
    return _tc_gin_node(h, aggr, p)


def kernel(x, edge_index, edge_attr, batch, params):
    src = edge_index[0]
    dst = edge_index[1]
    pad_e = _EP - _E
    src2 = jnp.concatenate(
        [src, jnp.zeros((pad_e,), jnp.int32)]).reshape(_ECH, 128)
    dst2 = jnp.concatenate(
        [dst, jnp.full((pad_e,), _N, jnp.int32)]).reshape(_ECH, 128)
    ea = jnp.pad(edge_attr, ((0, pad_e), (0, 0)))
    zeros = jnp.zeros((_ACC_R // 16, 64), jnp.float32)
    ones16 = jnp.ones((128, 16), jnp.float32)

    h = _tc_in(x, params['in_w'], params['in_b'].reshape(1, -1))
    h = _gin_layer(params['gin1_0'], h, ea, src2, dst2, zeros)
    h = _gin_layer(params['gin1_1'], h, ea, src2, dst2, zeros)
    h = _tc_bn_leaky(h, params['bn1_g'], params['bn1_b'])
    res = h

    tr = params['tr']
    q, kv = _tc_qkv(h, tr)
    kvg = _sc_gather128(kv, src2)
    qg = _sc_gather64(q, dst2)
    num_e, ue16 = _tc_tr_edge(ea, kvg, qg, tr)
    num = _sc_scatter64(num_e, dst2, zeros)
    den = _sc_scatter16(ue16, dst2, zeros)
    h = _tc_tr_node(h, num, den[0:_N], den[_N:2 * _N], tr)
    h = _tc_bn_leaky_res(h, params['bntr_g'], params['bntr_b'], res)

    h = _gin_layer(params['gin2_0'], h, ea, src2, dst2, zeros)
    h = _gin_layer(params['gin2_1'], h, ea, src2, dst2, zeros)
    h = _tc_bn_leaky(h, params['bn2_g'], params['bn2_b'])

    hp = jnp.pad(h, ((0, _NP - _N), (0, 0)))
    bp = jnp.concatenate(
        [batch, jnp.full((_NP - _N,), _G, jnp.int32)]).reshape(_NCH, 128)
    sums, cnts = _sc_pool(hp, bp, zeros, ones16)
    return _tc_cls(sums[0:_G], sums[_G:2 * _G], cnts[0:_G], cnts[_G:2 * _G],
                   params['cls_w1'], params['cls_b1'],
                   params['cls_w2'], params['cls_b2'])


# speedup vs baseline: 6.6135x; 6.6135x over previous
import functools
import math

import jax
import jax.numpy as jnp
from jax import lax
from jax.experimental import pallas as pl
from jax.experimental.pallas import tpu as pltpu
from jax.experimental.pallas import tpu_sc as plsc

_N = 50000
_E = 800000
_D = 64
_ED = 16
_G = 512

_EP = 819200
_ECH = _EP // 128
_BE = 2048
_BN = 2000
_ACC_R = 51200
_OPAD = 50048
_ORW = 3128
_NP = 65536
_NCH = _NP // 128

_mesh = plsc.VectorSubcoreMesh(core_axis_name="c", subcore_axis_name="s")


def _leaky(x):
    return jnp.where(x >= 0, x, 0.15 * x)


def _sigmoid(x):
    return 1.0 / (1.0 + jnp.exp(-x))


def _ln(x, g, b):
    m = jnp.mean(x, axis=-1, keepdims=True)
    v = jnp.mean((x - m) * (x - m), axis=-1, keepdims=True)
    return (x - m) / jnp.sqrt(v + 1e-5) * g + b


def _make_sc_gather(w):
    n_ch_tile = _ECH // 32
    n_sup = n_ch_tile // 8
    n_fire = 8 if w <= 64 else 4

    @functools.partial(
        pl.kernel,
        mesh=_mesh,
        compiler_params=pltpu.CompilerParams(use_tc_tiling_on_sc=False),
        out_type=jax.ShapeDtypeStruct((_EP, w), jnp.float32),
        scratch_types=[
            pltpu.VMEM((8, 128), jnp.int32),
            pltpu.VMEM((n_fire * 128, w), jnp.float32),
            pltpu.SemaphoreType.DMA,
        ],
    )
    def k(table_hbm, idx_hbm, out_hbm, idx_v, rows_v, sem):
        c = lax.axis_index("c")
        s = lax.axis_index("s")
        ch_base = (c * 16 + s) * n_ch_tile

        def body(sup, carry):
            ch0 = ch_base + sup * 8
            pltpu.sync_copy(idx_hbm.at[pl.ds(ch0, 8)], idx_v)
            for half in range(8 // n_fire):
                handles = [
                    pltpu.async_copy(
                        table_hbm.at[idx_v.at[half * n_fire + j]],
                        rows_v.at[pl.ds(j * 128, 128)],
                        sem,
                    )
                    for j in range(n_fire)
                ]
                for h in handles:
                    h.wait()
                pltpu.sync_copy(
                    rows_v,
                    out_hbm.at[pl.ds((ch0 + half * n_fire) * 128,
                                     n_fire * 128)],
                )
            return carry

        lax.fori_loop(0, n_sup, body, 0)

    return k


def _make_sc_scatter(split_cols):
    kb = 8
    n_ch_tile = _ECH // (16 if split_cols else 32)
    n_sup = n_ch_tile // kb
    n_pass = 2 if split_cols else 1
    zrows = _ACC_R // 16

    @functools.partial(
        pl.kernel,
        mesh=_mesh,
        compiler_params=pltpu.CompilerParams(use_tc_tiling_on_sc=False),
        out_type=jax.ShapeDtypeStruct(
            (2, _OPAD, 16 * n_pass), jnp.float32),
        scratch_types=[
            pltpu.VMEM((kb, 128), jnp.int32),
            pltpu.VMEM((kb * 128, 16), jnp.float32),
            pltpu.VMEM_SHARED((_ACC_R, 16), jnp.float32),
            pltpu.SemaphoreType.DMA,
        ],
    )
    def k(vals_hbm, idx_hbm, zeros_hbm, out_hbm, idx_v, vals_v, acc, sem):
        c = lax.axis_index("c")
        s = lax.axis_index("s")
        for p in range(n_pass):
            pltpu.sync_copy(
                zeros_hbm.at[pl.ds(0, zrows), pl.ds(0, 16)],
                acc.at[pl.ds(s * zrows, zrows)],
            )
            plsc.subcore_barrier()

            def body(sup, carry):
                if split_cols:
                    ch0 = s * n_ch_tile + sup * kb
                    pltpu.sync_copy(
                        vals_hbm.at[c, pl.ds(ch0 * 128, kb * 128),
                                    pl.ds(p * 16, 16)],
                        vals_v)
                else:
                    ch0 = (c * 16 + s) * n_ch_tile + sup * kb
                    pltpu.sync_copy(
                        vals_hbm.at[pl.ds(ch0 * 128, kb * 128)], vals_v)
                pltpu.sync_copy(idx_hbm.at[pl.ds(ch0, kb)], idx_v)
                handles = [
                    pltpu.async_copy(
                        vals_v.at[pl.ds(j * 128, 128)],
                        acc.at[idx_v.at[j]],
                        sem,
                        add=True,
                    )
                    for j in range(kb)
                ]
                for h in handles:
                    h.wait()
                return carry

            lax.fori_loop(0, n_sup, body, 0)
            plsc.subcore_barrier()
            if split_cols:
                pltpu.sync_copy(
                    acc.at[pl.ds(s * _ORW, _ORW)],
                    out_hbm.at[c, pl.ds(s * _ORW, _ORW), pl.ds(p * 16, 16)],
                )
            else:
                pltpu.sync_copy(
                    acc.at[pl.ds(s * _ORW, _ORW)],
                    out_hbm.at[c, pl.ds(s * _ORW, _ORW)],
                )
            if p + 1 < n_pass:
                plsc.subcore_barrier()

    return k


def _make_sc_pool():
    n_ch_tile = _NCH // 32
    gacc = _G + 16
    zrows = gacc // 16
    orows = _G // 16

    @functools.partial(
        pl.kernel,
        mesh=_mesh,
        compiler_params=pltpu.CompilerParams(use_tc_tiling_on_sc=False),
        out_type=[
            jax.ShapeDtypeStruct((2 * _G, 64), jnp.float32),
            jax.ShapeDtypeStruct((2 * _G, 16), jnp.float32),
        ],
        scratch_types=[
            pltpu.VMEM((8, 128), jnp.int32),
            pltpu.VMEM((8 * 128, 64), jnp.float32),
            pltpu.VMEM((128, 16), jnp.float32),
            pltpu.VMEM_SHARED((gacc, 64), jnp.float32),
            pltpu.VMEM_SHARED((gacc, 16), jnp.float32),
            pltpu.SemaphoreType.DMA,
        ],
    )
    def k(h_hbm, idx_hbm, zeros_hbm, ones_hbm, outh_hbm, outc_hbm,
          idx_v, vals_v, ones_v, acc_h, acc_c, sem):
        c = lax.axis_index("c")
        s = lax.axis_index("s")
        pltpu.sync_copy(ones_hbm, ones_v)
        pltpu.sync_copy(
            zeros_hbm.at[pl.ds(0, zrows), pl.ds(0, 64)],
            acc_h.at[pl.ds(s * zrows, zrows)],
        )
        pltpu.sync_copy(
            zeros_hbm.at[pl.ds(0, zrows), pl.ds(0, 16)],
            acc_c.at[pl.ds(s * zrows, zrows)],
        )
        plsc.subcore_barrier()

        for sup in range(n_ch_tile // 8):
            ch0 = (c * 16 + s) * n_ch_tile + sup * 8
            pltpu.sync_copy(idx_hbm.at[pl.ds(ch0, 8)], idx_v)
            pltpu.sync_copy(h_hbm.at[pl.ds(ch0 * 128, 8 * 128)], vals_v)
            handles = []
            for j in range(8):
                handles.append(pltpu.async_copy(
                    vals_v.at[pl.ds(j * 128, 128)],
                    acc_h.at[idx_v.at[j]],
                    sem,
                    add=True,
                ))
                handles.append(pltpu.async_copy(
                    ones_v,
                    acc_c.at[idx_v.at[j]],
                    sem,
                    add=True,
                ))
            for h in handles:
                h.wait()
        plsc.subcore_barrier()
        pltpu.sync_copy(
            acc_h.at[pl.ds(s * orows, orows)],
            outh_hbm.at[pl.ds(c * _G + s * orows, orows)],
        )
        pltpu.sync_copy(
            acc_c.at[pl.ds(s * orows, orows)],
            outc_hbm.at[pl.ds(c * _G + s * orows, orows)],
        )

    return k


_sc_gather64 = _make_sc_gather(64)
_sc_gather128 = _make_sc_gather(128)
_sc_scatter32x2 = _make_sc_scatter(True)
_sc_scatter16 = _make_sc_scatter(False)
_sc_pool = _make_sc_pool()


def _full_spec(shape):
    nd = len(shape)
    return pl.BlockSpec(shape, lambda i=0, _n=nd: (0,) * _n)


def _tc_in(x, w, b):
    def body(x_ref, w_ref, b_ref, o_ref):
        o_ref[...] = x_ref[...] * w_ref[...] + b_ref[...]

    return pl.pallas_call(
        body,
        grid=(_N // _BN,),
        in_specs=[
            pl.BlockSpec((_BN, 1), lambda i: (i, 0)),
            _full_spec((1, _D)),
            _full_spec((1, _D)),
        ],
        out_specs=pl.BlockSpec((_BN, _D), lambda i: (i, 0)),
        out_shape=jax.ShapeDtypeStruct((_N, _D), jnp.float32),
    )(x, w, b)


def _tc_gin_edge(ea, xg, p):
    def body(ea_ref, xg_ref, w1, b1, lg, lb, w2, b2, w3, b3, o_ref):
        e = jnp.dot(ea_ref[...], w1[...], preferred_element_type=jnp.float32)
        e = _ln(e + b1[...], lg[...], lb[...])
        e = _leaky(e)
        e = _leaky(jnp.dot(e, w2[...], preferred_element_type=jnp.float32) + b2[...])
        e = jnp.dot(e, w3[...], preferred_element_type=jnp.float32) + b3[...]
        gate = _sigmoid(e)
        msg = gate * xg_ref[...] + (1.0 - gate) * e
        o_ref[0] = msg[:, 0:32]
        o_ref[1] = msg[:, 32:64]

    enc = p['enc']
    return pl.pallas_call(
        body,
        grid=(_EP // _BE,),
        in_specs=[
            pl.BlockSpec((_BE, _ED), lambda i: (i, 0)),
            pl.BlockSpec((_BE, _D), lambda i: (i, 0)),
            _full_spec((_ED, _D)), _full_spec((1, _D)),
            _full_spec((1, _D)), _full_spec((1, _D)),
            _full_spec((_D, _D)), _full_spec((1, _D)),
            _full_spec((_D, _D)), _full_spec((1, _D)),
        ],
        out_specs=pl.BlockSpec((2, _BE, 32), lambda i: (0, i, 0)),
        out_shape=jax.ShapeDtypeStruct((2, _EP, 32), jnp.float32),
    )(ea, xg,
      enc['w1'], enc['b1'].reshape(1, -1),
      enc['ln_g'].reshape(1, -1), enc['ln_b'].reshape(1, -1),
      enc['w2'], enc['b2'].reshape(1, -1),
      enc['w3'], enc['b3'].reshape(1, -1))


def _tc_gin_node(h, aggr, p):
    def body(h_ref, a_ref, eps, w1, b1, l1g, l1b, w2, b2, l2g, l2b, o_ref):
        x = h_ref[...]
        a = jnp.concatenate([a_ref[0], a_ref[1]], axis=1)
        t = (1.0 + eps[0, 0]) * x + a
        t = jnp.dot(t, w1[...], preferred_element_type=jnp.float32) + b1[...]
        t = _leaky(_ln(t, l1g[...], l1b[...]))
        t = jnp.dot(t, w2[...], preferred_element_type=jnp.float32) + b2[...]
        t = _ln(t, l2g[...], l2b[...])
        o_ref[...] = x + t

    m = p['mlp']
    return pl.pallas_call(
        body,
        grid=(_N // _BN,),
        in_specs=[
            pl.BlockSpec((_BN, _D), lambda i: (i, 0)),
            pl.BlockSpec((2, _BN, 32), lambda i: (0, i, 0)),
            _full_spec((1, 1)),
            _full_spec((_D, 2 * _D)), _full_spec((1, 2 * _D)),
            _full_spec((1, 2 * _D)), _full_spec((1, 2 * _D)),
            _full_spec((2 * _D, _D)), _full_spec((1, _D)),
            _full_spec((1, _D)), _full_spec((1, _D)),
        ],
        out_specs=pl.BlockSpec((_BN, _D), lambda i: (i, 0)),
        out_shape=jax.ShapeDtypeStruct((_N, _D), jnp.float32),
    )(h, aggr, p['eps'].reshape(1, 1),
      m['w1'], m['b1'].reshape(1, -1), m['ln1_g'].reshape(1, -1),
      m['ln1_b'].reshape(1, -1), m['w2'], m['b2'].reshape(1, -1),
      m['ln2_g'].reshape(1, -1), m['ln2_b'].reshape(1, -1))


def _tc_bn_leaky(h, g, b, res=None):
    nb = _N // _BN
    with_res = res is not None

    def body(*refs):
        if with_res:
            h_ref, g_ref, b_ref, r_ref, o_ref, acc = refs
        else:
            h_ref, g_ref, b_ref, o_ref, acc = refs
        ph = pl.program_id(0)
        i = pl.program_id(1)

        @pl.when((ph == 0) & (i == 0))
        def _():
            acc[...] = jnp.zeros_like(acc)

        @pl.when(ph == 0)
        def _():
            x = h_ref[...]
            acc[0:1, :] += jnp.sum(x, axis=0, keepdims=True)
            acc[1:2, :] += jnp.sum(x * x, axis=0, keepdims=True)

        @pl.when(ph == 1)
        def _():
            x = h_ref[...]
            m = acc[0:1, :] * (1.0 / _N)
            v = acc[1:2, :] * (1.0 / _N) - m * m
            y = _leaky((x - m) / jnp.sqrt(v + 1e-5) * g_ref[...] + b_ref[...])
            if with_res:
                y = y + r_ref[...]
            o_ref[...] = y

    in_specs = [
        pl.BlockSpec((_BN, _D), lambda p, i: (i, 0)),
        pl.BlockSpec((1, _D), lambda p, i: (0, 0)),
        pl.BlockSpec((1, _D), lambda p, i: (0, 0)),
    ]
    args = [h, g.reshape(1, -1), b.reshape(1, -1)]
    if with_res:
        in_specs.append(pl.BlockSpec((_BN, _D), lambda p, i: (i, 0)))
        args.append(res)
    return pl.pallas_call(
        body,
        grid=(2, nb),
        in_specs=in_specs,
        out_specs=pl.BlockSpec((_BN, _D), lambda p, i: (i, 0)),
        out_shape=jax.ShapeDtypeStruct((_N, _D), jnp.float32),
        scratch_shapes=[pltpu.VMEM((8, _D), jnp.float32)],
    )(*args)


def _tc_bn_leaky_res(h, g, b, res):
    return _tc_bn_leaky(h, g, b, res)


def _tc_qkv(h, p):
    def body(h_ref, wq, bq, wk, bk, wv, bv, q_ref, kv_ref):
        x = h_ref[...]
        q_ref[...] = jnp.dot(x, wq[...], preferred_element_type=jnp.float32) + bq[...]
        kv_ref[:, 0:_D] = jnp.dot(x, wk[...], preferred_element_type=jnp.float32) + bk[...]
        kv_ref[:, _D:2 * _D] = jnp.dot(x, wv[...], preferred_element_type=jnp.float32) + bv[...]

    return pl.pallas_call(
        body,
        grid=(_N // _BN,),
        in_specs=[
            pl.BlockSpec((_BN, _D), lambda i: (i, 0)),
            _full_spec((_D, _D)), _full_spec((1, _D)),
            _full_spec((_D, _D)), _full_spec((1, _D)),
            _full_spec((_D, _D)), _full_spec((1, _D)),
        ],
        out_specs=[
            pl.BlockSpec((_BN, _D), lambda i: (i, 0)),
            pl.BlockSpec((_BN, 2 * _D), lambda i: (i, 0)),
        ],
        out_shape=[
            jax.ShapeDtypeStruct((_N, _D), jnp.float32),
            jax.ShapeDtypeStruct((_N, 2 * _D), jnp.float32),
        ],
    )(h, p['wq'], p['bq'].reshape(1, -1), p['wk'], p['bk'].reshape(1, -1),
      p['wv'], p['bv'].reshape(1, -1))


def _tc_tr_edge(ea, kvg, qg, p):
    isq = 1.0 / math.sqrt(_D // 2)

    def body(ea_ref, kvg_ref, qg_ref, w1, b1, w2, b2, we, be, num_ref, ue_ref):
        e = _leaky(jnp.dot(ea_ref[...], w1[...], preferred_element_type=jnp.float32) + b1[...])
        e = jnp.dot(e, w2[...], preferred_element_type=jnp.float32) + b2[...]
        ek = jnp.dot(e, we[...], preferred_element_type=jnp.float32) + be[...]
        kj = kvg_ref[:, 0:_D] + ek
        qk = qg_ref[...] * kj
        a0 = jnp.sum(qk[:, 0:32], axis=1, keepdims=True) * isq
        a1 = jnp.sum(qk[:, 32:64], axis=1, keepdims=True) * isq
        ue0 = jnp.exp(a0)
        ue1 = jnp.exp(a1)
        ve = kvg_ref[:, _D:2 * _D] + ek
        num_ref[0] = ve[:, 0:32] * ue0
        num_ref[1] = ve[:, 32:64] * ue1
        col16 = lax.broadcasted_iota(jnp.int32, (_BE, 16), 1)
        ue_ref[...] = jnp.where(col16 < 8, ue0, ue1)

    return pl.pallas_call(
        body,
        grid=(_EP // _BE,),
        in_specs=[
            pl.BlockSpec((_BE, _ED), lambda i: (i, 0)),
            pl.BlockSpec((_BE, 2 * _D), lambda i: (i, 0)),
            pl.BlockSpec((_BE, _D), lambda i: (i, 0)),
            _full_spec((_ED, _D)), _full_spec((1, _D)),
            _full_spec((_D, _D)), _full_spec((1, _D)),
            _full_spec((_D, _D)), _full_spec((1, _D)),
        ],
        out_specs=[
            pl.BlockSpec((2, _BE, 32), lambda i: (0, i, 0)),
            pl.BlockSpec((_BE, 16), lambda i: (i, 0)),
        ],
        out_shape=[
            jax.ShapeDtypeStruct((2, _EP, 32), jnp.float32),
            jax.ShapeDtypeStruct((_EP, 16), jnp.float32),
        ],
    )(ea, kvg, qg,
      p['enc_w1'], p['enc_b1'].reshape(1, -1),
      p['enc_w2'], p['enc_b2'].reshape(1, -1),
      p['we'], p['be'].reshape(1, -1))


def _tc_tr_node(h, num, den, p):
    wb = p['wbeta']

    def body(h_ref, n_ref, d_ref, wskip, bskip, wb0, wb1, wb2, o_ref):
        den = d_ref[0] + d_ref[1]
        d0 = den[:, 0:1]
        d1 = den[:, 8:9]
        out0 = n_ref[0] / (d0 + 1e-16)
        out1 = n_ref[1] / (d1 + 1e-16)
        out = jnp.concatenate([out0, out1], axis=1)
        x_r = jnp.dot(h_ref[...], wskip[...], preferred_element_type=jnp.float32) + bskip[...]
        bl = (jnp.dot(out, wb0[...], preferred_element_type=jnp.float32)
              + jnp.dot(x_r, wb1[...], preferred_element_type=jnp.float32)
              + jnp.dot(out - x_r, wb2[...], preferred_element_type=jnp.float32))
        beta = _sigmoid(bl)
        o_ref[...] = beta * x_r + (1.0 - beta) * out

    return pl.pallas_call(
        body,
        grid=(_N // _BN,),
        in_specs=[
            pl.BlockSpec((_BN, _D), lambda i: (i, 0)),
            pl.BlockSpec((2, _BN, 32), lambda i: (0, i, 0)),
            pl.BlockSpec((2, _BN, 16), lambda i: (0, i, 0)),
            _full_spec((_D, _D)), _full_spec((1, _D)),
            _full_spec((_D, 1)), _full_spec((_D, 1)), _full_spec((_D, 1)),
        ],
        out_specs=pl.BlockSpec((_BN, _D), lambda i: (i, 0)),
        out_shape=jax.ShapeDtypeStruct((_N, _D), jnp.float32),
    )(h, num, den, p['wskip'], p['bskip'].reshape(1, -1),
      wb[0:_D], wb[_D:2 * _D], wb[2 * _D:3 * _D])


def _tc_cls(sums, cnts, w1, b1, w2, b2):
    def body(s_ref, c_ref, w1r, b1r, w2r, b2r, o_ref):
        total = s_ref[pl.ds(0, _G)] + s_ref[pl.ds(_G, _G)]
        cnt = c_ref[pl.ds(0, _G), 0:1] + c_ref[pl.ds(_G, _G), 0:1]
        pooled = total / jnp.maximum(cnt, 1.0)
        z = _leaky(jnp.dot(pooled, w1r[...], preferred_element_type=jnp.float32) + b1r[...])
        o_ref[...] = jnp.dot(z, w2r[...], preferred_element_type=jnp.float32) + b2r[...]

    return pl.pallas_call(
        body,
        in_specs=[
            _full_spec((2 * _G, _D)), _full_spec((2 * _G, 16)),
            _full_spec((_D, _D // 2)), _full_spec((1, _D // 2)),
            _full_spec((_D // 2, 10)), _full_spec((1, 10)),
        ],
        out_specs=_full_spec((_G, 10)),
        out_shape=jax.ShapeDtypeStruct((_G, 10), jnp.float32),
    )(sums, cnts, w1, b1.reshape(1, -1), w2, b2.reshape(1, -1))


def _gin_layer(p, h, ea, src2, dst2, zeros):
    xg = _sc_gather64(h, src2)
    msg = _tc_gin_edge(ea, xg, p)
    aggr = _sc_scatter32x2(msg, dst2, zeros)
    return _tc_gin_node(h, aggr, p)


def kernel(x, edge_index, edge_attr, batch, params):
    src = edge_index[0]
    dst = edge_index[1]
    pad_e = _EP - _E
    src2 = jnp.concatenate(
        [src, jnp.zeros((pad_e,), jnp.int32)]).reshape(_ECH, 128)
    dst2 = jnp.concatenate(
        [dst, jnp.full((pad_e,), _N, jnp.int32)]).reshape(_ECH, 128)
    ea = jnp.pad(edge_attr, ((0, pad_e), (0, 0)))
    zeros = jnp.zeros((_ACC_R // 16, 64), jnp.float32)
    ones16 = jnp.ones((128, 16), jnp.float32)

    h = _tc_in(x, params['in_w'], params['in_b'].reshape(1, -1))
    h = _gin_layer(params['gin1_0'], h, ea, src2, dst2, zeros)
    h = _gin_layer(params['gin1_1'], h, ea, src2, dst2, zeros)
    h = _tc_bn_leaky(h, params['bn1_g'], params['bn1_b'])
    res = h

    tr = params['tr']
    q, kv = _tc_qkv(h, tr)
    kvg = _sc_gather128(kv, src2)
    qg = _sc_gather64(q, dst2)
    num_e, ue16 = _tc_tr_edge(ea, kvg, qg, tr)
    num = _sc_scatter32x2(num_e, dst2, zeros)
    den = _sc_scatter16(ue16, dst2, zeros)
    h = _tc_tr_node(h, num, den, tr)
    h = _tc_bn_leaky_res(h, params['bntr_g'], params['bntr_b'], res)

    h = _gin_layer(params['gin2_0'], h, ea, src2, dst2, zeros)
    h = _gin_layer(params['gin2_1'], h, ea, src2, dst2, zeros)
    h = _tc_bn_leaky(h, params['bn2_g'], params['bn2_b'])

    hp = jnp.pad(h, ((0, _NP - _N), (0, 0)))
    bp = jnp.concatenate(
        [batch, jnp.full((_NP - _N,), _G, jnp.int32)]).reshape(_NCH, 128)
    sums, cnts = _sc_pool(hp, bp, zeros, ones16)
    return _tc_cls(sums, cnts,
                   params['cls_w1'], params['cls_b1'],
                   params['cls_w2'], params['cls_b2'])
